# baseline (speedup 1.0000x reference)
import functools
import math

import jax
import jax.numpy as jnp
from jax import lax
from jax.experimental import pallas as pl
from jax.experimental.pallas import tpu as pltpu

_INV_SQRT_2 = 0.7071067811865475


def _round_up(x, m):
    return -(-x // m) * m


def _ffn_kernel(x_ref, w1a_ref, w1b_ref, b1_ref, w2a_ref, w2b_ref, b2_ref,
                o_ref, acc_ref, sem):
    i = pl.program_id(0)
    j = pl.program_id(1)
    ni = pl.num_programs(0)
    nj = pl.num_programs(1)
    tm = acc_ref.shape[0]
    hh = w1a_ref.shape[1]

    @pl.when((j == 0) & (i > 0))
    def _():
        pltpu.make_async_copy(
            acc_ref, o_ref.at[pl.ds((i - 1) * tm, tm), :], sem).wait()

    ha = jnp.dot(x_ref[...], w1a_ref[...], preferred_element_type=jnp.float32)
    ha = ha + b1_ref[:, :hh]
    ha = 0.5 * ha * (1.0 + lax.erf(ha * _INV_SQRT_2))
    hb = jnp.dot(x_ref[...], w1b_ref[...], preferred_element_type=jnp.float32)
    hb = hb + b1_ref[:, hh:]
    hb = 0.5 * hb * (1.0 + lax.erf(hb * _INV_SQRT_2))
    first = j == 0
    D = o_ref.shape[1]
    nc = D // 4
    for d in range(4):
        dsl = slice(d * nc, (d + 1) * nc)
        y = (jnp.dot(ha, w2a_ref[:, dsl], preferred_element_type=jnp.float32)
             + jnp.dot(hb, w2b_ref[:, dsl],
                       preferred_element_type=jnp.float32))
        acc_ref[:, dsl] = jnp.where(first, y + b2_ref[:, dsl],
                                    acc_ref[:, dsl] + y)

    @pl.when(j == nj - 1)
    def _():
        pltpu.make_async_copy(
            acc_ref, o_ref.at[pl.ds(i * tm, tm), :], sem).start()

    @pl.when((j == nj - 1) & (i == ni - 1))
    def _():
        pltpu.make_async_copy(
            acc_ref, o_ref.at[pl.ds(i * tm, tm), :], sem).wait()


@functools.partial(jax.jit, static_argnames=("tm", "th"))
def _ffn(x, w1, b1, w2, b2, *, tm=1024, th=1024):
    orig_shape = x.shape
    D = orig_shape[-1]
    H = w1.shape[1]
    M = math.prod(orig_shape[:-1])

    tm = max(8, min(tm, _round_up(M, 8)))
    Mp = _round_up(M, tm)
    xb = x.reshape(M, D)
    if Mp != M:
        xb = jnp.pad(xb, ((0, Mp - M), (0, 0)))

    while H % th:
        th //= 2

    grid = (Mp // tm, H // th)
    b1_2d = b1.reshape(1, H)
    b2_2d = b2.reshape(1, D)

    w_item = jnp.dtype(w1.dtype).itemsize
    cost = pl.CostEstimate(
        flops=4 * Mp * D * H,
        transcendentals=Mp * H,
        bytes_accessed=(Mp * D * (2 + 4)
                        + grid[0] * (2 * D * H + H) * w_item + D * w_item),
    )

    out2d = pl.pallas_call(
        _ffn_kernel,
        out_shape=jax.ShapeDtypeStruct((Mp, D), jnp.float32),
        grid=grid,
        in_specs=[
            pl.BlockSpec((tm, D), lambda i, j: (i, 0)),
            pl.BlockSpec((D, th // 2), lambda i, j: (0, 2 * j)),
            pl.BlockSpec((D, th // 2), lambda i, j: (0, 2 * j + 1)),
            pl.BlockSpec((1, th), lambda i, j: (0, j)),
            pl.BlockSpec((th // 2, D), lambda i, j: (2 * j, 0)),
            pl.BlockSpec((th // 2, D), lambda i, j: (2 * j + 1, 0)),
            pl.BlockSpec((1, D), lambda i, j: (0, 0)),
        ],
        out_specs=pl.BlockSpec(memory_space=pl.ANY),
        scratch_shapes=[
            pltpu.VMEM((tm, D), jnp.float32),
            pltpu.SemaphoreType.DMA,
        ],
        compiler_params=pltpu.CompilerParams(
            dimension_semantics=("arbitrary", "arbitrary"),
            vmem_limit_bytes=63 * 1024 * 1024,
        ),
        cost_estimate=cost,
    )(xb, w1, w1, b1_2d, w2, w2, b2_2d)

    if Mp != M:
        out2d = out2d[:M]
    return out2d.reshape(orig_shape)


def kernel(x, w1, b1, w2, b2):
    return _ffn(x, w1, b1, w2, b2)


# speedup vs baseline: 1.0089x; 1.0089x over previous
import functools
import math

import jax
import jax.numpy as jnp
from jax import lax
from jax.experimental import pallas as pl
from jax.experimental.pallas import tpu as pltpu

_INV_SQRT_2 = 0.7071067811865475


def _round_up(x, m):
    return -(-x // m) * m


def _ffn_kernel(x_ref, w1_ref, b1_ref, w2_ref, b2_ref, o_ref, acc_ref, sem):
    i = pl.program_id(0)
    j = pl.program_id(1)
    ni = pl.num_programs(0)
    nj = pl.num_programs(1)
    tm = acc_ref.shape[0]

    h = jnp.dot(x_ref[...], w1_ref[...],
                preferred_element_type=jnp.float32)
    h = h + b1_ref[...]
    h = 0.5 * h * (1.0 + lax.erf(h * _INV_SQRT_2))

    @pl.when((j == 0) & (i > 0))
    def _():
        pltpu.make_async_copy(
            acc_ref, o_ref.at[pl.ds((i - 1) * tm, tm), :], sem).wait()
    first = j == 0
    D = o_ref.shape[1]
    nc = D // 4
    for d in range(4):
        dsl = slice(d * nc, (d + 1) * nc)
        y = jnp.dot(h, w2_ref[:, dsl], preferred_element_type=jnp.float32)
        acc_ref[:, dsl] = jnp.where(first, y + b2_ref[:, dsl],
                                    acc_ref[:, dsl] + y)

    @pl.when(j == nj - 1)
    def _():
        pltpu.make_async_copy(
            acc_ref, o_ref.at[pl.ds(i * tm, tm), :], sem).start()

    @pl.when((j == nj - 1) & (i == ni - 1))
    def _():
        pltpu.make_async_copy(
            acc_ref, o_ref.at[pl.ds(i * tm, tm), :], sem).wait()


@functools.partial(jax.jit, static_argnames=("tm", "th"))
def _ffn(x, w1, b1, w2, b2, *, tm=1024, th=1024):
    orig_shape = x.shape
    D = orig_shape[-1]
    H = w1.shape[1]
    M = math.prod(orig_shape[:-1])

    tm = max(8, min(tm, _round_up(M, 8)))
    Mp = _round_up(M, tm)
    xb = x.reshape(M, D)
    if Mp != M:
        xb = jnp.pad(xb, ((0, Mp - M), (0, 0)))

    while H % th:
        th //= 2

    grid = (Mp // tm, H // th)
    b1_2d = b1.reshape(1, H)
    b2_2d = b2.reshape(1, D)

    w_item = jnp.dtype(w1.dtype).itemsize
    cost = pl.CostEstimate(
        flops=4 * Mp * D * H,
        transcendentals=Mp * H,
        bytes_accessed=(Mp * D * (2 + 4)
                        + grid[0] * (2 * D * H + H) * w_item + D * w_item),
    )

    out2d = pl.pallas_call(
        _ffn_kernel,
        out_shape=jax.ShapeDtypeStruct((Mp, D), jnp.float32),
        grid=grid,
        in_specs=[
            pl.BlockSpec((tm, D), lambda i, j: (i, 0)),
            pl.BlockSpec((D, th), lambda i, j: (0, j)),
            pl.BlockSpec((1, th), lambda i, j: (0, j)),
            pl.BlockSpec((th, D), lambda i, j: (j, 0)),
            pl.BlockSpec((1, D), lambda i, j: (0, 0)),
        ],
        out_specs=pl.BlockSpec(memory_space=pl.ANY),
        scratch_shapes=[
            pltpu.VMEM((tm, D), jnp.float32),
            pltpu.SemaphoreType.DMA,
        ],
        compiler_params=pltpu.CompilerParams(
            dimension_semantics=("arbitrary", "arbitrary"),
            vmem_limit_bytes=63 * 1024 * 1024,
        ),
        cost_estimate=cost,
    )(xb, w1, b1_2d, w2, b2_2d)

    if Mp != M:
        out2d = out2d[:M]
    return out2d.reshape(orig_shape)


def kernel(x, w1, b1, w2, b2):
    return _ffn(x, w1, b1, w2, b2)
